# baseline (speedup 1.0000x reference)
import functools

import jax
import jax.numpy as jnp
from jax import lax
from jax.experimental import pallas as pl
from jax.experimental.pallas import tpu as pltpu
from jax.experimental.pallas import tpu_sc as plsc

_NC = 2
_NS = 16
_NW = _NC * _NS
_LANES = 16
_CH = 8


def _sc_body(B, S, D, x_hbm, pos_hbm, out_hbm,
             pb, xb0, xb1, xb2, xb3, spm, si0, si1, so0, so1, sx0, sx1):
    sid = lax.axis_index("s")
    wid = sid * _NC + lax.axis_index("c")
    s_per_w = S // _NW
    nchunk = s_per_w // _CH
    base = wid * s_per_w
    xbs = (xb0, xb1, xb2, xb3)
    sin = (si0, si1)
    sout = (so0, so1)
    sxb = (sx0, sx1)

    def start_in(c):
        par = c % 2
        s0 = base + c * _CH
        hs = [pltpu.make_async_copy(pos_hbm.at[pl.ds(s0, _CH)],
                                    pb.at[par], sin[par])]
        for b in range(B):
            hs.append(pltpu.make_async_copy(x_hbm.at[pl.ds(b * S + s0, _CH)],
                                            xbs[b].at[par], sin[par]))
        for h in hs:
            h.start()
        return hs

    def start_xbar(c):
        par = c % 2
        hs = []
        for b in range(B):
            hs.append(pltpu.make_async_copy(xbs[b].at[par],
                                            spm.at[sid, b], sxb[par]))
        for h in hs:
            h.start()
        return hs

    def start_out(c):
        par = c % 2
        s0 = base + c * _CH
        hs = []
        for b in range(B):
            hs.append(pltpu.make_async_copy(spm.at[sid, b],
                                            out_hbm.at[pl.ds(b * S + s0, _CH)],
                                            sout[par]))
        for h in hs:
            h.start()
        return hs

    ncol = D // _LANES

    def compute(par):
        @plsc.parallel_loop(0, _CH * ncol, unroll=8)
        def slice_add(j):
            i = j // ncol
            col = j % ncol
            sl = pl.ds(col * _LANES, _LANES)
            pv = pb[par, i, sl]
            for b in range(B):
                plsc.addupdate(xbs[b].at[par, i, sl], pv)

    in_h = {0: start_in(0)}
    xbar_h = {}
    out_h = {}

    def drain(table, c):
        if c in table:
            for h in table.pop(c):
                h.wait()

    for c in range(nchunk):
        if c >= 1:
            drain(xbar_h, c - 1)
            out_h[c - 1] = start_out(c - 1)
        if c + 1 < nchunk:
            in_h[c + 1] = start_in(c + 1)
        drain(in_h, c)
        compute(c % 2)
        drain(out_h, c - 1)
        xbar_h[c] = start_xbar(c)
    drain(xbar_h, nchunk - 1)
    out_h[nchunk - 1] = start_out(nchunk - 1)
    drain(out_h, nchunk - 2)
    drain(out_h, nchunk - 1)


def kernel(x, pos_table):
    B, S, D = x.shape
    xf = x.reshape(B * S, D)

    mesh = plsc.VectorSubcoreMesh(core_axis_name="c", subcore_axis_name="s")
    sc_add = pl.kernel(
        functools.partial(_sc_body, B, S, D),
        out_type=jax.ShapeDtypeStruct((B * S, D), jnp.float32),
        mesh=mesh,
        scratch_types=[
            pltpu.VMEM((2, _CH, D), jnp.float32),
            pltpu.VMEM((2, _CH, D), jnp.float32),
            pltpu.VMEM((2, _CH, D), jnp.float32),
            pltpu.VMEM((2, _CH, D), jnp.float32),
            pltpu.VMEM((2, _CH, D), jnp.float32),
            pltpu.VMEM_SHARED((_NS, B, _CH, D), jnp.float32),
            pltpu.SemaphoreType.DMA,
            pltpu.SemaphoreType.DMA,
            pltpu.SemaphoreType.DMA,
            pltpu.SemaphoreType.DMA,
            pltpu.SemaphoreType.DMA,
            pltpu.SemaphoreType.DMA,
        ],
    )
    out = sc_add(xf, pos_table)
    return out.reshape(B, S, D)


# speedup vs baseline: 1.0268x; 1.0268x over previous
import functools

import jax
import jax.numpy as jnp
from jax import lax
from jax.experimental import pallas as pl
from jax.experimental.pallas import tpu as pltpu
from jax.experimental.pallas import tpu_sc as plsc

_NC = 2
_NS = 16
_NW = _NC * _NS
_LANES = 16
_CH = 8


def _sc_body(B, S, D, x_hbm, pos_hbm, out_hbm,
             pb, xb0, xb1, xb2, xb3, si0, si1, so0, so1):
    wid = lax.axis_index("s") * _NC + lax.axis_index("c")
    s_per_w = S // _NW
    nchunk = s_per_w // _CH
    base = wid * s_per_w
    xbs = (xb0, xb1, xb2, xb3)
    sin = (si0, si1)
    sout = (so0, so1)

    def start_in(c):
        par = c % 2
        s0 = base + c * _CH
        hs = [pltpu.make_async_copy(pos_hbm.at[pl.ds(s0, _CH)],
                                    pb.at[par], sin[par])]
        for b in range(B):
            hs.append(pltpu.make_async_copy(x_hbm.at[pl.ds(b * S + s0, _CH)],
                                            xbs[b].at[par], sin[par]))
        for h in hs:
            h.start()
        return hs

    def start_out(c):
        par = c % 2
        s0 = base + c * _CH
        hs = []
        for b in range(B):
            hs.append(pltpu.make_async_copy(xbs[b].at[par],
                                            out_hbm.at[pl.ds(b * S + s0, _CH)],
                                            sout[par]))
        for h in hs:
            h.start()
        return hs

    ncol = D // _LANES

    def compute(par):
        @plsc.parallel_loop(0, _CH * ncol, unroll=8)
        def slice_add(j):
            i = j // ncol
            col = j % ncol
            sl = pl.ds(col * _LANES, _LANES)
            pv = pb[par, i, sl]
            for b in range(B):
                plsc.addupdate(xbs[b].at[par, i, sl], pv)

    in_h = {0: start_in(0)}
    out_h = {}
    for c in range(nchunk):
        if c + 1 < nchunk:
            if c - 1 >= 0:
                for h in out_h.pop(c - 1):
                    h.wait()
            in_h[c + 1] = start_in(c + 1)
        for h in in_h.pop(c):
            h.wait()
        compute(c % 2)
        out_h[c] = start_out(c)
    for c in (nchunk - 2, nchunk - 1):
        if c >= 0 and c in out_h:
            for h in out_h.pop(c):
                h.wait()


def kernel(x, pos_table):
    B, S, D = x.shape
    xf = x.reshape(B * S, D)

    mesh = plsc.VectorSubcoreMesh(core_axis_name="c", subcore_axis_name="s")
    sc_add = pl.kernel(
        functools.partial(_sc_body, B, S, D),
        out_type=jax.ShapeDtypeStruct((B * S, D), jnp.float32),
        mesh=mesh,
        scratch_types=[
            pltpu.VMEM((2, _CH, D), jnp.float32),
            pltpu.VMEM((2, _CH, D), jnp.float32),
            pltpu.VMEM((2, _CH, D), jnp.float32),
            pltpu.VMEM((2, _CH, D), jnp.float32),
            pltpu.VMEM((2, _CH, D), jnp.float32),
            pltpu.SemaphoreType.DMA,
            pltpu.SemaphoreType.DMA,
            pltpu.SemaphoreType.DMA,
            pltpu.SemaphoreType.DMA,
        ],
    )
    out = sc_add(xf, pos_table)
    return out.reshape(B, S, D)
